# baseline (speedup 1.0000x reference)
import functools

import jax
import jax.numpy as jnp
from jax import lax
from jax.experimental import pallas as pl
from jax.experimental.pallas import tpu as pltpu
from jax.experimental.pallas import tpu_sc as plsc

BN_EPS = 1e-5
B = 4
N = 8192
NP = 2048
KNN = 16
CIN = 64
COUT = 64
RK, LC = 64, 128


def _fps_body(coords_ref, idx_ref):
    gidx = (lax.broadcasted_iota(jnp.int32, (RK, LC), 0) * LC
            + lax.broadcasted_iota(jnp.int32, (RK, LC), 1))
    igid = (lax.broadcasted_iota(jnp.int32, (16, LC), 0) * LC
            + lax.broadcasted_iota(jnp.int32, (16, LC), 1))

    def step(i, state):
        far, dists, acc = state
        newfar = []
        newdists = []
        newacc = []
        for b in range(B):
            f = far[b]
            m = gidx == f
            newacc.append(jnp.where(igid == i, f, acc[b]))
            X = coords_ref[b, 0]
            Y = coords_ref[b, 1]
            Z = coords_ref[b, 2]
            cx = jnp.sum(jnp.where(m, X, 0.0), keepdims=True).reshape(1, 1)
            cy = jnp.sum(jnp.where(m, Y, 0.0), keepdims=True).reshape(1, 1)
            cz = jnp.sum(jnp.where(m, Z, 0.0), keepdims=True).reshape(1, 1)
            dx = X - cx
            dy = Y - cy
            dz = Z - cz
            nd = jnp.minimum(dists[b], (dx * dx + dy * dy) + dz * dz)
            mx = jnp.max(nd, keepdims=True).reshape(1, 1)
            nf = jnp.min(jnp.where(nd == mx, gidx, jnp.int32(2**30)),
                         keepdims=True).reshape(1, 1)
            newfar.append(nf)
            newdists.append(nd)
        return tuple(newfar), tuple(newdists), tuple(newacc)

    init = (tuple(jnp.zeros((1, 1), jnp.int32) for _ in range(B)),
            tuple(jnp.full((RK, LC), 1e10, jnp.float32) for _ in range(B)),
            tuple(jnp.zeros((16, LC), jnp.int32) for _ in range(B)))
    _, _, acc = lax.fori_loop(0, NP, step, init, unroll=False)
    for b in range(B):
        idx_ref[b] = acc[b]


def _fps(coords):
    return pl.pallas_call(
        _fps_body,
        out_shape=jax.ShapeDtypeStruct((B, 16, LC), jnp.int32),
    )(coords)


TS = 256


def _knn_body(xyzT_ref, new_ref, w3_ref, idx_ref, q_ref, D_ref):
    X1 = xyzT_ref[0, 0:1, :]
    Y1 = xyzT_ref[0, 1:2, :]
    Z1 = xyzT_ref[0, 2:3, :]
    nx = new_ref[0, :, 0:1]
    ny = new_ref[0, :, 1:2]
    nz = new_ref[0, :, 2:3]
    sx = (X1 * X1 + Y1 * Y1) + Z1 * Z1
    sn = (nx * nx + ny * ny) + nz * nz
    dot = jnp.dot(new_ref[0], xyzT_ref[0], preferred_element_type=jnp.float32)
    D_ref[...] = (sn - 2.0 * dot) + sx
    lane = lax.broadcasted_iota(jnp.int32, (TS, N), 1)
    BIGI = jnp.int32(2**30)
    am = jnp.full((TS, 1), -1, jnp.int32)
    for k in range(KNN):
        Dcur = jnp.where(lane == am, 1e30, D_ref[...])
        D_ref[...] = Dcur
        mval = jnp.min(Dcur, axis=1, keepdims=True)
        am = jnp.min(jnp.where(Dcur == mval, lane, BIGI), axis=1,
                     keepdims=True)
        idx_ref[0, :, k:k + 1] = am
    q_ref[0] = (nx * w3_ref[0:1, :] + ny * w3_ref[1:2, :]) + nz * w3_ref[2:3, :]


def _knn(xyzT, new_xyz, w3):
    grid = (B, NP // TS)
    return pl.pallas_call(
        _knn_body,
        grid=grid,
        in_specs=[
            pl.BlockSpec((1, 3, N), lambda b, s: (b, 0, 0)),
            pl.BlockSpec((1, TS, 3), lambda b, s: (b, s, 0)),
            pl.BlockSpec((3, COUT), lambda b, s: (0, 0)),
        ],
        out_specs=[
            pl.BlockSpec((1, TS, KNN), lambda b, s: (b, s, 0)),
            pl.BlockSpec((1, TS, COUT), lambda b, s: (b, s, 0)),
        ],
        out_shape=[
            jax.ShapeDtypeStruct((B, NP, KNN), jnp.int32),
            jax.ShapeDtypeStruct((B, NP, COUT), jnp.float32),
        ],
        scratch_shapes=[pltpu.VMEM((TS, N), jnp.float32)],
    )(xyzT, new_xyz, w3)


TG = 2048


def _gmat_body(feat_ref, xyz_ref, w64_ref, w3_ref, g_ref):
    f = feat_ref[0]
    g = jnp.dot(f, w64_ref[...], preferred_element_type=jnp.float32,
                precision=lax.Precision.HIGHEST)
    x0 = xyz_ref[0][:, 0:1]
    x1 = xyz_ref[0][:, 1:2]
    x2 = xyz_ref[0][:, 2:3]
    g = g + ((x0 * w3_ref[0:1, :] + x1 * w3_ref[1:2, :])
             + x2 * w3_ref[2:3, :])
    g_ref[0] = jnp.concatenate(
        [g, xyz_ref[0], jnp.zeros((TG, COUT - 3), jnp.float32)], axis=1)


def _gmat(features, xyz, w64, w3):
    grid = (B, N // TG)
    return pl.pallas_call(
        _gmat_body,
        grid=grid,
        in_specs=[
            pl.BlockSpec((1, TG, CIN), lambda b, s: (b, s, 0)),
            pl.BlockSpec((1, TG, 3), lambda b, s: (b, s, 0)),
            pl.BlockSpec((CIN, COUT), lambda b, s: (0, 0)),
            pl.BlockSpec((3, COUT), lambda b, s: (0, 0)),
        ],
        out_specs=pl.BlockSpec((1, TG, 2 * COUT), lambda b, s: (b, s, 0)),
        out_shape=jax.ShapeDtypeStruct((B, N, 2 * COUT), jnp.float32),
    )(features, xyz, w64, w3)


_NC = 2
_NS = 16
_NW = _NC * _NS
_GTOT = B * NP * KNN
_NTOT = B * NP


@functools.cache
def _sc_gather_fn(n_rows, chunk):
    per_w = n_rows // _NW
    n_ch = per_w // chunk

    def body(table_hbm, idx_hbm, out_hbm, idx_v, rows_v, sem):
        wid = lax.axis_index("s") * _NC + lax.axis_index("c")
        base = wid * per_w
        for c in range(n_ch):
            off = base + c * chunk
            pltpu.sync_copy(idx_hbm.at[pl.ds(off, chunk)], idx_v)
            pltpu.async_copy(table_hbm.at[idx_v], rows_v, sem).wait()
            pltpu.sync_copy(rows_v, out_hbm.at[pl.ds(off, chunk)])

    return pl.kernel(
        body,
        out_type=jax.ShapeDtypeStruct((n_rows, 2 * COUT), jnp.float32),
        mesh=plsc.VectorSubcoreMesh(core_axis_name="c", subcore_axis_name="s",
                                    num_cores=_NC, num_subcores=_NS),
        scratch_types=[
            pltpu.VMEM((chunk,), jnp.int32),
            pltpu.VMEM((chunk, 2 * COUT), jnp.float32),
            pltpu.SemaphoreType.DMA,
        ],
    )


def _sc_gather(table, idx, n_rows, chunk):
    return _sc_gather_fn(n_rows, chunk)(table, idx)


TR = 256


def _pool_body(ggrp_ref, q_ref, mmax_ref, mmin_ref, sums_ref):
    first = (pl.program_id(0) == 0) & (pl.program_id(1) == 0)
    g = ggrp_ref[0].reshape(TR, KNN, 2 * COUT)[:, :, :COUT]
    h = g - q_ref[0][:, None, :]
    mmax_ref[0] = jnp.max(h, axis=1)
    mmin_ref[0] = jnp.min(h, axis=1)
    s1 = jnp.sum(h, axis=(0, 1))
    s2 = jnp.sum(h * h, axis=(0, 1))
    s = jnp.concatenate([s1[None, :], s2[None, :]], axis=0)
    sums_ref[...] = jnp.where(first, s, sums_ref[...] + s)


def _pool(ggrp, q):
    grid = (B, NP // TR)
    return pl.pallas_call(
        _pool_body,
        grid=grid,
        in_specs=[
            pl.BlockSpec((1, TR * KNN, 2 * COUT), lambda b, s: (b, s, 0)),
            pl.BlockSpec((1, TR, COUT), lambda b, s: (b, s, 0)),
        ],
        out_specs=[
            pl.BlockSpec((1, TR, COUT), lambda b, s: (b, s, 0)),
            pl.BlockSpec((1, TR, COUT), lambda b, s: (b, s, 0)),
            pl.BlockSpec((2, COUT), lambda b, s: (0, 0)),
        ],
        out_shape=[
            jax.ShapeDtypeStruct((B, NP, COUT), jnp.float32),
            jax.ShapeDtypeStruct((B, NP, COUT), jnp.float32),
            jax.ShapeDtypeStruct((2, COUT), jnp.float32),
        ],
    )(ggrp, q)


def _bn_body(sums_ref, gamma_ref, beta_ref, mmax_ref, mmin_ref, out_ref):
    cnt = float(B * NP * KNN)
    mean = sums_ref[0:1, :] / cnt
    var = sums_ref[1:2, :] / cnt - mean * mean
    std = jnp.sqrt(var + BN_EPS)
    gam = gamma_ref[...]
    sel = jnp.where(gam >= 0, mmax_ref[0], mmin_ref[0])
    out_ref[0] = jnp.maximum((sel - mean) / std * gam + beta_ref[...], 0.0)


def _bn(sums, gamma, beta, mmax, mmin):
    grid = (B, NP // TR)
    return pl.pallas_call(
        _bn_body,
        grid=grid,
        in_specs=[
            pl.BlockSpec((2, COUT), lambda b, s: (0, 0)),
            pl.BlockSpec((1, COUT), lambda b, s: (0, 0)),
            pl.BlockSpec((1, COUT), lambda b, s: (0, 0)),
            pl.BlockSpec((1, TR, COUT), lambda b, s: (b, s, 0)),
            pl.BlockSpec((1, TR, COUT), lambda b, s: (b, s, 0)),
        ],
        out_specs=pl.BlockSpec((1, TR, COUT), lambda b, s: (b, s, 0)),
        out_shape=jax.ShapeDtypeStruct((B, NP, COUT), jnp.float32),
    )(sums, gamma, beta, mmax, mmin)


def kernel(xyz, features, W, gamma, beta):
    xyzT = xyz.transpose(0, 2, 1)
    coords = xyzT.reshape(B, 3, RK, LC)
    w3 = W[:3]
    w64 = W[3:]

    fps_idx = _fps(coords).reshape(B, NP)
    g = _gmat(features, xyz, w64, w3)
    table = g.reshape(B * N, 2 * COUT)
    offs = (jnp.arange(B, dtype=jnp.int32) * N)[:, None]
    new_rows = _sc_gather(table, (fps_idx + offs).reshape(_NTOT), _NTOT, 256)
    new_xyz = new_rows[:, COUT:COUT + 3].reshape(B, NP, 3)
    knn_idx, q = _knn(xyzT, new_xyz, w3)
    flat_idx = (knn_idx.reshape(B, NP * KNN) + offs).reshape(_GTOT)
    ggrp = _sc_gather(table, flat_idx, _GTOT, 512)
    mmax, mmin, sums = _pool(ggrp.reshape(B, NP * KNN, 2 * COUT), q)
    out = _bn(sums, gamma.reshape(1, COUT), beta.reshape(1, COUT), mmax, mmin)
    return (new_xyz, out)


# speedup vs baseline: 1.3235x; 1.3235x over previous
import functools

import jax
import jax.numpy as jnp
from jax import lax
from jax.experimental import pallas as pl
from jax.experimental.pallas import tpu as pltpu
from jax.experimental.pallas import tpu_sc as plsc

BN_EPS = 1e-5
B = 4
N = 8192
NP = 2048
KNN = 16
CIN = 64
COUT = 64
RK, LC = 64, 128


def _fps_body(coords_ref, xyz_smem_ref, newx_ref):
    gidx = (lax.broadcasted_iota(jnp.int32, (RK, LC), 0) * LC
            + lax.broadcasted_iota(jnp.int32, (RK, LC), 1))

    def step(i, state):
        far, dists = state
        cents = []
        for b in range(B):
            f = far[b]
            cents.append((xyz_smem_ref[3 * b, f],
                          xyz_smem_ref[3 * b + 1, f],
                          xyz_smem_ref[3 * b + 2, f]))
        newfar = []
        newdists = []
        for b in range(B):
            cx, cy, cz = cents[b]
            newx_ref[3 * b, i] = cx
            newx_ref[3 * b + 1, i] = cy
            newx_ref[3 * b + 2, i] = cz
            dx = coords_ref[b, 0] - cx
            dy = coords_ref[b, 1] - cy
            dz = coords_ref[b, 2] - cz
            nd = jnp.minimum(dists[b], (dx * dx + dy * dy) + dz * dz)
            mx = jnp.max(nd, keepdims=True).reshape(1, 1)
            nf = jnp.min(jnp.where(nd == mx, gidx, jnp.int32(2**30)))
            newfar.append(nf)
            newdists.append(nd)
        return tuple(newfar), tuple(newdists)

    init = (tuple(jnp.int32(0) for _ in range(B)),
            tuple(jnp.full((RK, LC), 1e10, jnp.float32) for _ in range(B)))
    lax.fori_loop(0, NP, step, init, unroll=False)


def _fps(coords, xyz_smem):
    return pl.pallas_call(
        _fps_body,
        in_specs=[
            pl.BlockSpec(memory_space=pltpu.VMEM),
            pl.BlockSpec(memory_space=pltpu.SMEM),
        ],
        out_specs=pl.BlockSpec(memory_space=pltpu.SMEM),
        out_shape=jax.ShapeDtypeStruct((B * 3, NP), jnp.float32),
    )(coords, xyz_smem)


TS = 256


def _knn_body(xyzT_ref, new_ref, w3_ref, idx_ref, q_ref, D_ref):
    X1 = xyzT_ref[0, 0:1, :]
    Y1 = xyzT_ref[0, 1:2, :]
    Z1 = xyzT_ref[0, 2:3, :]
    nx = new_ref[0, :, 0:1]
    ny = new_ref[0, :, 1:2]
    nz = new_ref[0, :, 2:3]
    sx = (X1 * X1 + Y1 * Y1) + Z1 * Z1
    sn = (nx * nx + ny * ny) + nz * nz
    dot = jnp.dot(new_ref[0], xyzT_ref[0], preferred_element_type=jnp.float32)
    D_ref[...] = (sn - 2.0 * dot) + sx
    lane = lax.broadcasted_iota(jnp.int32, (TS, N), 1)
    BIGI = jnp.int32(2**30)
    am = jnp.full((TS, 1), -1, jnp.int32)
    for k in range(KNN):
        Dcur = jnp.where(lane == am, 1e30, D_ref[...])
        D_ref[...] = Dcur
        mval = jnp.min(Dcur, axis=1, keepdims=True)
        am = jnp.min(jnp.where(Dcur == mval, lane, BIGI), axis=1,
                     keepdims=True)
        idx_ref[0, :, k:k + 1] = am
    q_ref[0] = (nx * w3_ref[0:1, :] + ny * w3_ref[1:2, :]) + nz * w3_ref[2:3, :]


def _knn(xyzT, new_xyz, w3):
    grid = (B, NP // TS)
    return pl.pallas_call(
        _knn_body,
        grid=grid,
        in_specs=[
            pl.BlockSpec((1, 3, N), lambda b, s: (b, 0, 0)),
            pl.BlockSpec((1, TS, 3), lambda b, s: (b, s, 0)),
            pl.BlockSpec((3, COUT), lambda b, s: (0, 0)),
        ],
        out_specs=[
            pl.BlockSpec((1, TS, KNN), lambda b, s: (b, s, 0)),
            pl.BlockSpec((1, TS, COUT), lambda b, s: (b, s, 0)),
        ],
        out_shape=[
            jax.ShapeDtypeStruct((B, NP, KNN), jnp.int32),
            jax.ShapeDtypeStruct((B, NP, COUT), jnp.float32),
        ],
        scratch_shapes=[pltpu.VMEM((TS, N), jnp.float32)],
    )(xyzT, new_xyz, w3)


TG = 2048


def _gmat_body(feat_ref, xyz_ref, w64_ref, w3_ref, g_ref):
    f = feat_ref[0]
    g = jnp.dot(f, w64_ref[...], preferred_element_type=jnp.float32,
                precision=lax.Precision.HIGHEST)
    x0 = xyz_ref[0][:, 0:1]
    x1 = xyz_ref[0][:, 1:2]
    x2 = xyz_ref[0][:, 2:3]
    g = g + ((x0 * w3_ref[0:1, :] + x1 * w3_ref[1:2, :])
             + x2 * w3_ref[2:3, :])
    g_ref[0] = jnp.concatenate(
        [g, xyz_ref[0], jnp.zeros((TG, COUT - 3), jnp.float32)], axis=1)


def _gmat(features, xyz, w64, w3):
    grid = (B, N // TG)
    return pl.pallas_call(
        _gmat_body,
        grid=grid,
        in_specs=[
            pl.BlockSpec((1, TG, CIN), lambda b, s: (b, s, 0)),
            pl.BlockSpec((1, TG, 3), lambda b, s: (b, s, 0)),
            pl.BlockSpec((CIN, COUT), lambda b, s: (0, 0)),
            pl.BlockSpec((3, COUT), lambda b, s: (0, 0)),
        ],
        out_specs=pl.BlockSpec((1, TG, 2 * COUT), lambda b, s: (b, s, 0)),
        out_shape=jax.ShapeDtypeStruct((B, N, 2 * COUT), jnp.float32),
    )(features, xyz, w64, w3)


_NC = 2
_NS = 16
_NW = _NC * _NS
_GTOT = B * NP * KNN
_NTOT = B * NP


@functools.cache
def _sc_gather_fn(n_rows, chunk):
    per_w = n_rows // _NW
    n_ch = per_w // chunk

    def body(table_hbm, idx_hbm, out_hbm, idx_v, rows_v, sem):
        wid = lax.axis_index("s") * _NC + lax.axis_index("c")
        base = wid * per_w
        for c in range(n_ch):
            off = base + c * chunk
            pltpu.sync_copy(idx_hbm.at[pl.ds(off, chunk)], idx_v)
            pltpu.async_copy(table_hbm.at[idx_v], rows_v, sem).wait()
            pltpu.sync_copy(rows_v, out_hbm.at[pl.ds(off, chunk)])

    return pl.kernel(
        body,
        out_type=jax.ShapeDtypeStruct((n_rows, 2 * COUT), jnp.float32),
        mesh=plsc.VectorSubcoreMesh(core_axis_name="c", subcore_axis_name="s",
                                    num_cores=_NC, num_subcores=_NS),
        scratch_types=[
            pltpu.VMEM((chunk,), jnp.int32),
            pltpu.VMEM((chunk, 2 * COUT), jnp.float32),
            pltpu.SemaphoreType.DMA,
        ],
    )


def _sc_gather(table, idx, n_rows, chunk):
    return _sc_gather_fn(n_rows, chunk)(table, idx)


TR = 256


def _pool_body(ggrp_ref, q_ref, mmax_ref, mmin_ref, sums_ref):
    first = (pl.program_id(0) == 0) & (pl.program_id(1) == 0)
    g = ggrp_ref[0].reshape(TR, KNN, 2 * COUT)[:, :, :COUT]
    h = g - q_ref[0][:, None, :]
    mmax_ref[0] = jnp.max(h, axis=1)
    mmin_ref[0] = jnp.min(h, axis=1)
    s1 = jnp.sum(h, axis=(0, 1))
    s2 = jnp.sum(h * h, axis=(0, 1))
    s = jnp.concatenate([s1[None, :], s2[None, :]], axis=0)
    sums_ref[...] = jnp.where(first, s, sums_ref[...] + s)


def _pool(ggrp, q):
    grid = (B, NP // TR)
    return pl.pallas_call(
        _pool_body,
        grid=grid,
        in_specs=[
            pl.BlockSpec((1, TR * KNN, 2 * COUT), lambda b, s: (b, s, 0)),
            pl.BlockSpec((1, TR, COUT), lambda b, s: (b, s, 0)),
        ],
        out_specs=[
            pl.BlockSpec((1, TR, COUT), lambda b, s: (b, s, 0)),
            pl.BlockSpec((1, TR, COUT), lambda b, s: (b, s, 0)),
            pl.BlockSpec((2, COUT), lambda b, s: (0, 0)),
        ],
        out_shape=[
            jax.ShapeDtypeStruct((B, NP, COUT), jnp.float32),
            jax.ShapeDtypeStruct((B, NP, COUT), jnp.float32),
            jax.ShapeDtypeStruct((2, COUT), jnp.float32),
        ],
    )(ggrp, q)


def _bn_body(sums_ref, gamma_ref, beta_ref, mmax_ref, mmin_ref, out_ref):
    cnt = float(B * NP * KNN)
    mean = sums_ref[0:1, :] / cnt
    var = sums_ref[1:2, :] / cnt - mean * mean
    std = jnp.sqrt(var + BN_EPS)
    gam = gamma_ref[...]
    sel = jnp.where(gam >= 0, mmax_ref[0], mmin_ref[0])
    out_ref[0] = jnp.maximum((sel - mean) / std * gam + beta_ref[...], 0.0)


def _bn(sums, gamma, beta, mmax, mmin):
    grid = (B, NP // TR)
    return pl.pallas_call(
        _bn_body,
        grid=grid,
        in_specs=[
            pl.BlockSpec((2, COUT), lambda b, s: (0, 0)),
            pl.BlockSpec((1, COUT), lambda b, s: (0, 0)),
            pl.BlockSpec((1, COUT), lambda b, s: (0, 0)),
            pl.BlockSpec((1, TR, COUT), lambda b, s: (b, s, 0)),
            pl.BlockSpec((1, TR, COUT), lambda b, s: (b, s, 0)),
        ],
        out_specs=pl.BlockSpec((1, TR, COUT), lambda b, s: (b, s, 0)),
        out_shape=jax.ShapeDtypeStruct((B, NP, COUT), jnp.float32),
    )(sums, gamma, beta, mmax, mmin)


def kernel(xyz, features, W, gamma, beta):
    xyzT = xyz.transpose(0, 2, 1)
    coords = xyzT.reshape(B, 3, RK, LC)
    w3 = W[:3]
    w64 = W[3:]

    newx = _fps(coords, xyzT.reshape(B * 3, N))
    new_xyz = newx.reshape(B, 3, NP).transpose(0, 2, 1)
    g = _gmat(features, xyz, w64, w3)
    table = g.reshape(B * N, 2 * COUT)
    offs = (jnp.arange(B, dtype=jnp.int32) * N)[:, None]
    knn_idx, q = _knn(xyzT, new_xyz, w3)
    flat_idx = (knn_idx.reshape(B, NP * KNN) + offs).reshape(_GTOT)
    ggrp = _sc_gather(table, flat_idx, _GTOT, 512)
    mmax, mmin, sums = _pool(ggrp.reshape(B, NP * KNN, 2 * COUT), q)
    out = _bn(sums, gamma.reshape(1, COUT), beta.reshape(1, COUT), mmax, mmin)
    return (new_xyz, out)


# speedup vs baseline: 1.4032x; 1.0603x over previous
import functools

import jax
import jax.numpy as jnp
from jax import lax
from jax.experimental import pallas as pl
from jax.experimental.pallas import tpu as pltpu
from jax.experimental.pallas import tpu_sc as plsc

BN_EPS = 1e-5
B = 4
N = 8192
NP = 2048
KNN = 16
CIN = 64
COUT = 64
RK, LC = 64, 128


def _fps_body(coords_ref, xyz_smem_ref, newx_ref):
    gidx = (lax.broadcasted_iota(jnp.int32, (RK, LC), 0) * LC
            + lax.broadcasted_iota(jnp.int32, (RK, LC), 1))

    def step(i, state):
        far, dists = state
        cents = []
        for b in range(B):
            f = far[b]
            cents.append((xyz_smem_ref[3 * b, f],
                          xyz_smem_ref[3 * b + 1, f],
                          xyz_smem_ref[3 * b + 2, f]))
        newfar = []
        newdists = []
        for b in range(B):
            cx, cy, cz = cents[b]
            newx_ref[3 * b, i] = cx
            newx_ref[3 * b + 1, i] = cy
            newx_ref[3 * b + 2, i] = cz
            dx = coords_ref[b, 0] - cx
            dy = coords_ref[b, 1] - cy
            dz = coords_ref[b, 2] - cz
            nd = jnp.minimum(dists[b], (dx * dx + dy * dy) + dz * dz)
            mx = jnp.max(nd, keepdims=True).reshape(1, 1)
            nf = jnp.min(jnp.where(nd == mx, gidx, jnp.int32(2**30)))
            newfar.append(nf)
            newdists.append(nd)
        return tuple(newfar), tuple(newdists)

    init = (tuple(jnp.int32(0) for _ in range(B)),
            tuple(jnp.full((RK, LC), 1e10, jnp.float32) for _ in range(B)))
    lax.fori_loop(0, NP, step, init, unroll=4)


def _fps(coords, xyz_smem):
    return pl.pallas_call(
        _fps_body,
        in_specs=[
            pl.BlockSpec(memory_space=pltpu.VMEM),
            pl.BlockSpec(memory_space=pltpu.SMEM),
        ],
        out_specs=pl.BlockSpec(memory_space=pltpu.SMEM),
        out_shape=jax.ShapeDtypeStruct((B * 3, NP), jnp.float32),
    )(coords, xyz_smem)


TS = 256


def _knn_body(xyzT_ref, new_ref, w3_ref, idx_ref, q_ref, D_ref):
    X1 = xyzT_ref[0, 0:1, :]
    Y1 = xyzT_ref[0, 1:2, :]
    Z1 = xyzT_ref[0, 2:3, :]
    nx = new_ref[0, :, 0:1]
    ny = new_ref[0, :, 1:2]
    nz = new_ref[0, :, 2:3]
    sx = (X1 * X1 + Y1 * Y1) + Z1 * Z1
    sn = (nx * nx + ny * ny) + nz * nz
    dot = jnp.dot(new_ref[0], xyzT_ref[0], preferred_element_type=jnp.float32)
    D_ref[...] = (sn - 2.0 * dot) + sx
    lane = lax.broadcasted_iota(jnp.int32, (TS, N), 1)
    BIGI = jnp.int32(2**30)
    am = jnp.full((TS, 1), -1, jnp.int32)
    for k in range(KNN):
        Dcur = jnp.where(lane == am, 1e30, D_ref[...])
        D_ref[...] = Dcur
        mval = jnp.min(Dcur, axis=1, keepdims=True)
        am = jnp.min(jnp.where(Dcur == mval, lane, BIGI), axis=1,
                     keepdims=True)
        idx_ref[0, :, k:k + 1] = am
    q_ref[0] = (nx * w3_ref[0:1, :] + ny * w3_ref[1:2, :]) + nz * w3_ref[2:3, :]


def _knn(xyzT, new_xyz, w3):
    grid = (B, NP // TS)
    return pl.pallas_call(
        _knn_body,
        grid=grid,
        in_specs=[
            pl.BlockSpec((1, 3, N), lambda b, s: (b, 0, 0)),
            pl.BlockSpec((1, TS, 3), lambda b, s: (b, s, 0)),
            pl.BlockSpec((3, COUT), lambda b, s: (0, 0)),
        ],
        out_specs=[
            pl.BlockSpec((1, TS, KNN), lambda b, s: (b, s, 0)),
            pl.BlockSpec((1, TS, COUT), lambda b, s: (b, s, 0)),
        ],
        out_shape=[
            jax.ShapeDtypeStruct((B, NP, KNN), jnp.int32),
            jax.ShapeDtypeStruct((B, NP, COUT), jnp.float32),
        ],
        scratch_shapes=[pltpu.VMEM((TS, N), jnp.float32)],
    )(xyzT, new_xyz, w3)


TG = 2048


def _gmat_body(feat_ref, xyz_ref, w64_ref, w3_ref, g_ref):
    f = feat_ref[0]
    g = jnp.dot(f, w64_ref[...], preferred_element_type=jnp.float32,
                precision=lax.Precision.HIGHEST)
    x0 = xyz_ref[0][:, 0:1]
    x1 = xyz_ref[0][:, 1:2]
    x2 = xyz_ref[0][:, 2:3]
    g = g + ((x0 * w3_ref[0:1, :] + x1 * w3_ref[1:2, :])
             + x2 * w3_ref[2:3, :])
    g_ref[0] = jnp.concatenate(
        [g, xyz_ref[0], jnp.zeros((TG, COUT - 3), jnp.float32)], axis=1)


def _gmat(features, xyz, w64, w3):
    grid = (B, N // TG)
    return pl.pallas_call(
        _gmat_body,
        grid=grid,
        in_specs=[
            pl.BlockSpec((1, TG, CIN), lambda b, s: (b, s, 0)),
            pl.BlockSpec((1, TG, 3), lambda b, s: (b, s, 0)),
            pl.BlockSpec((CIN, COUT), lambda b, s: (0, 0)),
            pl.BlockSpec((3, COUT), lambda b, s: (0, 0)),
        ],
        out_specs=pl.BlockSpec((1, TG, 2 * COUT), lambda b, s: (b, s, 0)),
        out_shape=jax.ShapeDtypeStruct((B, N, 2 * COUT), jnp.float32),
    )(features, xyz, w64, w3)


_NC = 2
_NS = 16
_NW = _NC * _NS
_GTOT = B * NP * KNN
_NTOT = B * NP


@functools.cache
def _sc_gather_fn(n_rows, chunk):
    per_w = n_rows // _NW
    n_ch = per_w // chunk

    def body(table_hbm, idx_hbm, out_hbm, idx_v, rows_v, sem):
        wid = lax.axis_index("s") * _NC + lax.axis_index("c")
        base = wid * per_w
        for c in range(n_ch):
            off = base + c * chunk
            pltpu.sync_copy(idx_hbm.at[pl.ds(off, chunk)], idx_v)
            pltpu.async_copy(table_hbm.at[idx_v], rows_v, sem).wait()
            pltpu.sync_copy(rows_v, out_hbm.at[pl.ds(off, chunk)])

    return pl.kernel(
        body,
        out_type=jax.ShapeDtypeStruct((n_rows, 2 * COUT), jnp.float32),
        mesh=plsc.VectorSubcoreMesh(core_axis_name="c", subcore_axis_name="s",
                                    num_cores=_NC, num_subcores=_NS),
        scratch_types=[
            pltpu.VMEM((chunk,), jnp.int32),
            pltpu.VMEM((chunk, 2 * COUT), jnp.float32),
            pltpu.SemaphoreType.DMA,
        ],
    )


def _sc_gather(table, idx, n_rows, chunk):
    return _sc_gather_fn(n_rows, chunk)(table, idx)


TR = 256


def _pool_body(ggrp_ref, q_ref, mmax_ref, mmin_ref, sums_ref):
    first = (pl.program_id(0) == 0) & (pl.program_id(1) == 0)
    g = ggrp_ref[0].reshape(TR, KNN, 2 * COUT)[:, :, :COUT]
    h = g - q_ref[0][:, None, :]
    mmax_ref[0] = jnp.max(h, axis=1)
    mmin_ref[0] = jnp.min(h, axis=1)
    s1 = jnp.sum(h, axis=(0, 1))
    s2 = jnp.sum(h * h, axis=(0, 1))
    s = jnp.concatenate([s1[None, :], s2[None, :]], axis=0)
    sums_ref[...] = jnp.where(first, s, sums_ref[...] + s)


def _pool(ggrp, q):
    grid = (B, NP // TR)
    return pl.pallas_call(
        _pool_body,
        grid=grid,
        in_specs=[
            pl.BlockSpec((1, TR * KNN, 2 * COUT), lambda b, s: (b, s, 0)),
            pl.BlockSpec((1, TR, COUT), lambda b, s: (b, s, 0)),
        ],
        out_specs=[
            pl.BlockSpec((1, TR, COUT), lambda b, s: (b, s, 0)),
            pl.BlockSpec((1, TR, COUT), lambda b, s: (b, s, 0)),
            pl.BlockSpec((2, COUT), lambda b, s: (0, 0)),
        ],
        out_shape=[
            jax.ShapeDtypeStruct((B, NP, COUT), jnp.float32),
            jax.ShapeDtypeStruct((B, NP, COUT), jnp.float32),
            jax.ShapeDtypeStruct((2, COUT), jnp.float32),
        ],
    )(ggrp, q)


def _bn_body(sums_ref, gamma_ref, beta_ref, mmax_ref, mmin_ref, out_ref):
    cnt = float(B * NP * KNN)
    mean = sums_ref[0:1, :] / cnt
    var = sums_ref[1:2, :] / cnt - mean * mean
    std = jnp.sqrt(var + BN_EPS)
    gam = gamma_ref[...]
    sel = jnp.where(gam >= 0, mmax_ref[0], mmin_ref[0])
    out_ref[0] = jnp.maximum((sel - mean) / std * gam + beta_ref[...], 0.0)


def _bn(sums, gamma, beta, mmax, mmin):
    grid = (B, NP // TR)
    return pl.pallas_call(
        _bn_body,
        grid=grid,
        in_specs=[
            pl.BlockSpec((2, COUT), lambda b, s: (0, 0)),
            pl.BlockSpec((1, COUT), lambda b, s: (0, 0)),
            pl.BlockSpec((1, COUT), lambda b, s: (0, 0)),
            pl.BlockSpec((1, TR, COUT), lambda b, s: (b, s, 0)),
            pl.BlockSpec((1, TR, COUT), lambda b, s: (b, s, 0)),
        ],
        out_specs=pl.BlockSpec((1, TR, COUT), lambda b, s: (b, s, 0)),
        out_shape=jax.ShapeDtypeStruct((B, NP, COUT), jnp.float32),
    )(sums, gamma, beta, mmax, mmin)


def kernel(xyz, features, W, gamma, beta):
    xyzT = xyz.transpose(0, 2, 1)
    coords = xyzT.reshape(B, 3, RK, LC)
    w3 = W[:3]
    w64 = W[3:]

    newx = _fps(coords, xyzT.reshape(B * 3, N))
    new_xyz = newx.reshape(B, 3, NP).transpose(0, 2, 1)
    g = _gmat(features, xyz, w64, w3)
    table = g.reshape(B * N, 2 * COUT)
    offs = (jnp.arange(B, dtype=jnp.int32) * N)[:, None]
    knn_idx, q = _knn(xyzT, new_xyz, w3)
    flat_idx = (knn_idx.reshape(B, NP * KNN) + offs).reshape(_GTOT)
    ggrp = _sc_gather(table, flat_idx, _GTOT, 512)
    mmax, mmin, sums = _pool(ggrp.reshape(B, NP * KNN, 2 * COUT), q)
    out = _bn(sums, gamma.reshape(1, COUT), beta.reshape(1, COUT), mmax, mmin)
    return (new_xyz, out)
